# baseline (speedup 1.0000x reference)
import functools

import jax
import jax.numpy as jnp
import numpy as np
from jax import lax
from jax.experimental import pallas as pl
from jax.experimental.pallas import tpu as pltpu
from jax.experimental.pallas import tpu_sc as plsc

V = 100000
D = 32
DC = 4 * D
B = 1024
M = 50
S = 20
HOPS = 3

NW = 32
G_TOT = B * M + B
GPW = G_TOT // NW
G_CHUNK = 8
CHUNK_IDX = G_CHUNK * S
NCHUNK = GPW // G_CHUNK
NCH2 = NCHUNK // 2


def _position_encoding(sentence_size, embedding_dim):
    i = np.arange(1, embedding_dim + 1, dtype=np.float32)[:, None]
    j = np.arange(1, sentence_size + 1, dtype=np.float32)[None, :]
    enc = (i - (embedding_dim + 1) / 2.0) * (j - (sentence_size + 1) / 2.0)
    enc = 1.0 + 4.0 * enc / embedding_dim / sentence_size
    return np.transpose(enc).astype(np.float32)


_ENC_CAT = np.tile(_position_encoding(S, D), (1, 4))

_PERM = np.concatenate([np.arange(0, D, 2), np.arange(1, D, 2)])
_INV_PERM = np.argsort(_PERM)
_ENC_PERM = _position_encoding(S, D)[:, _PERM]


def _sc_weighted_gather(t0, t1, t2, t3, idx_all, enc):
    mesh = plsc.VectorSubcoreMesh(core_axis_name="c", subcore_axis_name="s")
    HALF = CHUNK_IDX // 2
    W = D // 2

    @functools.partial(
        pl.kernel,
        out_type=jax.ShapeDtypeStruct((G_TOT, DC), jnp.float32),
        mesh=mesh,
        compiler_params=pltpu.CompilerParams(
            use_tc_tiling_on_sc=False, needs_layout_passes=False),
        scratch_types=[
            pltpu.VMEM((GPW * S,), jnp.int32),
            pltpu.VMEM((2, 4, CHUNK_IDX, W), jnp.float32),
            pltpu.VMEM((2, G_CHUNK, DC), jnp.float32),
            pltpu.VMEM((S, D), jnp.float32),
            pltpu.SemaphoreType.DMA,
            pltpu.SemaphoreType.DMA,
            pltpu.SemaphoreType.DMA,
            pltpu.SemaphoreType.DMA,
        ],
    )
    def k(t0h, t1h, t2h, t3h, idx_hbm, enc_hbm, out_hbm,
          idx_v, rows_v, out_v, enc_v, sem0, sem1, semo0, semo1):
        tabs = (t0h, t1h, t2h, t3h)
        sems = (sem0, sem1)
        semos = (semo0, semo1)
        wid = lax.axis_index("s") * 2 + lax.axis_index("c")
        pltpu.sync_copy(enc_hbm, enc_v)
        pltpu.sync_copy(idx_hbm.at[pl.ds(wid * GPW * S, GPW * S)], idx_v)

        def issue(buf, c):
            off = c * CHUNK_IDX
            for t in range(4):
                for h in range(2):
                    pltpu.async_copy(
                        tabs[t].at[idx_v.at[pl.ds(off + h * HALF, HALF)]],
                        rows_v.at[buf, t, pl.ds(h * HALF, HALF)], sems[buf])

        def drain(buf):
            for t in range(4):
                for h in range(2):
                    pltpu.make_async_copy(
                        tabs[t].at[pl.ds(0, HALF)],
                        rows_v.at[buf, t, pl.ds(h * HALF, HALF)],
                        sems[buf]).wait()

        def drain_out(buf):
            pltpu.make_async_copy(
                out_v.at[buf], out_hbm.at[pl.ds(0, G_CHUNK)],
                semos[buf]).wait()

        def compute_and_write(buf, c):
            for t in range(4):
                e_lo = [enc_v[s, pl.ds(0, 16)] for s in range(S)]
                e_hi = [enc_v[s, pl.ds(16, 16)] for s in range(S)]

                def g_body(g, _):
                    def term(s):
                        w = rows_v[buf, t, g * S + s, :]
                        return plsc.unpack(
                            plsc.bitcast(w, jnp.bfloat16),
                            format=plsc.PackFormat.INTERLEAVED)

                    lo, hi = term(0)
                    acc_lo = lo * e_lo[0]
                    acc_hi = hi * e_hi[0]
                    for s in range(1, S):
                        lo, hi = term(s)
                        acc_lo = acc_lo + lo * e_lo[s]
                        acc_hi = acc_hi + hi * e_hi[s]
                    out_v[buf, g, pl.ds(t * D, 16)] = acc_lo
                    out_v[buf, g, pl.ds(t * D + 16, 16)] = acc_hi
                    return 0

                lax.fori_loop(0, G_CHUNK, g_body, 0, unroll=False)
            gbase = wid * GPW + c * G_CHUNK
            pltpu.async_copy(out_v.at[buf],
                             out_hbm.at[pl.ds(gbase, G_CHUNK)], semos[buf])

        issue(0, 0)

        def loop_body(cc, _):
            c0 = 2 * cc
            issue(1, c0 + 1)
            drain(0)

            @pl.when(cc > 0)
            def _():
                drain_out(0)

            compute_and_write(0, c0)

            @pl.when(cc + 1 < NCH2)
            def _():
                issue(0, c0 + 2)

            drain(1)

            @pl.when(cc > 0)
            def _():
                drain_out(1)

            compute_and_write(1, c0 + 1)
            return 0

        lax.fori_loop(0, NCH2, loop_body, 0, unroll=False)
        drain_out(0)
        drain_out(1)

    return k(t0, t1, t2, t3, idx_all, enc)


def _hops(m_cat, u0):
    BB = 256

    def body(m_ref, u0_ref, u_ref):
        u = u0_ref[...]
        mc = m_ref[...]
        for h in range(HOPS):
            mk = mc[:, :, h * D:(h + 1) * D]
            scores = jnp.sum(mk * u[:, None, :], axis=2)
            smax = jnp.max(scores, axis=1, keepdims=True)
            e = jnp.exp(scores - smax)
            p = e / jnp.sum(e, axis=1, keepdims=True)
            mk1 = mc[:, :, (h + 1) * D:(h + 2) * D]
            u = u + jnp.sum(mk1 * p[:, :, None], axis=1)
        u_ref[...] = u

    return pl.pallas_call(
        body,
        grid=(B // BB,),
        in_specs=[
            pl.BlockSpec((BB, M, DC), lambda i: (i, 0, 0)),
            pl.BlockSpec((BB, D), lambda i: (i, 0)),
        ],
        out_specs=pl.BlockSpec((BB, D), lambda i: (i, 0)),
        out_shape=jax.ShapeDtypeStruct((B, D), jnp.float32),
    )(m_cat, u0)


VSTRIP = 2000
NSTRIP = V // VSTRIP


def _projT_stats(uT, c3):
    def body(c3_ref, uT_ref, ahatT_ref, stat_ref, acc_m, acc_s):
        v = pl.program_id(0)

        @pl.when(v == 0)
        def _():
            acc_m[...] = jnp.full_like(acc_m, -jnp.inf)
            acc_s[...] = jnp.zeros_like(acc_s)

        l = jnp.dot(c3_ref[...], uT_ref[...],
                    preferred_element_type=jnp.float32)
        ahatT_ref[...] = l
        bm = jnp.max(l, axis=0, keepdims=True)
        m_old = acc_m[...]
        m_new = jnp.maximum(m_old, bm)
        acc_s[...] = (acc_s[...] * jnp.exp(m_old - m_new)
                      + jnp.sum(jnp.exp(l - m_new), axis=0, keepdims=True))
        acc_m[...] = m_new

        @pl.when(v == NSTRIP - 1)
        def _():
            stat_ref[0:1, :] = acc_m[...]
            stat_ref[1:2, :] = 1.0 / acc_s[...]

    return pl.pallas_call(
        body,
        grid=(NSTRIP,),
        in_specs=[
            pl.BlockSpec((VSTRIP, D), lambda i: (i, 0)),
            pl.BlockSpec((D, B), lambda i: (0, 0)),
        ],
        out_specs=[
            pl.BlockSpec((VSTRIP, B), lambda i: (i, 0)),
            pl.BlockSpec((8, B), lambda i: (0, 0)),
        ],
        out_shape=[
            jax.ShapeDtypeStruct((V, B), jnp.float32),
            jax.ShapeDtypeStruct((8, B), jnp.float32),
        ],
        scratch_shapes=[
            pltpu.VMEM((1, B), jnp.float32),
            pltpu.VMEM((1, B), jnp.float32),
        ],
    )(c3, uT)


def _projT_write(uT, c3, stats):
    def body(c3_ref, uT_ref, stat_ref, softT_ref):
        l = jnp.dot(c3_ref[...], uT_ref[...],
                    preferred_element_type=jnp.float32)
        m = stat_ref[0:1, :]
        inv_s = stat_ref[1:2, :]
        softT_ref[...] = jnp.exp(l - m) * inv_s

    return pl.pallas_call(
        body,
        grid=(NSTRIP,),
        in_specs=[
            pl.BlockSpec((VSTRIP, D), lambda i: (i, 0)),
            pl.BlockSpec((D, B), lambda i: (0, 0)),
            pl.BlockSpec((8, B), lambda i: (0, 0)),
        ],
        out_specs=pl.BlockSpec((VSTRIP, B), lambda i: (i, 0)),
        out_shape=jax.ShapeDtypeStruct((V, B), jnp.float32),
    )(c3, uT, stats)


def _pack_table(c):
    cb = c.astype(jnp.bfloat16).reshape(V, D // 2, 2)
    return jax.lax.bitcast_convert_type(cb, jnp.float32)


def kernel(story, query, C0, C1, C2, C3):
    idx_all = jnp.concatenate(
        [story.reshape(-1), query.reshape(-1)]).astype(jnp.int32)
    enc = jnp.asarray(_ENC_PERM)

    m_all = _sc_weighted_gather(
        _pack_table(C0), _pack_table(C1), _pack_table(C2), _pack_table(C3),
        idx_all, enc)
    m_cat = m_all[:B * M].reshape(B, M, DC)
    u0 = m_all[B * M:, :D]

    u = _hops(m_cat, u0)
    u = u[:, jnp.asarray(_INV_PERM)]
    uT = u.T.astype(jnp.bfloat16)
    c3b = C3.astype(jnp.bfloat16)
    ahatT, stats = _projT_stats(uT, c3b)
    softT = _projT_write(uT, c3b, stats)
    return ahatT.T, softT.T


# speedup vs baseline: 1.2553x; 1.2553x over previous
import functools

import jax
import jax.numpy as jnp
import numpy as np
from jax import lax
from jax.experimental import pallas as pl
from jax.experimental.pallas import tpu as pltpu
from jax.experimental.pallas import tpu_sc as plsc

V = 100000
D = 32
DC = 4 * D
B = 1024
M = 50
S = 20
HOPS = 3

NW = 32
G_TOT = B * M + B
GPW = G_TOT // NW
G_CHUNK = 8
CHUNK_IDX = G_CHUNK * S
NCHUNK = GPW // G_CHUNK
NCH2 = NCHUNK // 2


def _position_encoding(sentence_size, embedding_dim):
    i = np.arange(1, embedding_dim + 1, dtype=np.float32)[:, None]
    j = np.arange(1, sentence_size + 1, dtype=np.float32)[None, :]
    enc = (i - (embedding_dim + 1) / 2.0) * (j - (sentence_size + 1) / 2.0)
    enc = 1.0 + 4.0 * enc / embedding_dim / sentence_size
    return np.transpose(enc).astype(np.float32)


_ENC_CAT = np.tile(_position_encoding(S, D), (1, 4))

_PERM = np.concatenate([np.arange(0, D, 2), np.arange(1, D, 2)])
_INV_PERM = np.argsort(_PERM)
_ENC_PERM = _position_encoding(S, D)[:, _PERM]


def _sc_weighted_gather(t0, t1, t2, t3, idx_all, enc):
    mesh = plsc.VectorSubcoreMesh(core_axis_name="c", subcore_axis_name="s")
    HALF = CHUNK_IDX // 2
    W = D // 2

    @functools.partial(
        pl.kernel,
        out_type=jax.ShapeDtypeStruct((G_TOT, DC), jnp.float32),
        mesh=mesh,
        compiler_params=pltpu.CompilerParams(
            use_tc_tiling_on_sc=False, needs_layout_passes=False),
        scratch_types=[
            pltpu.VMEM((GPW * S,), jnp.int32),
            pltpu.VMEM((2, 4, CHUNK_IDX, W), jnp.float32),
            pltpu.VMEM((2, G_CHUNK, DC), jnp.float32),
            pltpu.VMEM((S, D), jnp.float32),
            pltpu.SemaphoreType.DMA,
            pltpu.SemaphoreType.DMA,
            pltpu.SemaphoreType.DMA,
            pltpu.SemaphoreType.DMA,
        ],
    )
    def k(t0h, t1h, t2h, t3h, idx_hbm, enc_hbm, out_hbm,
          idx_v, rows_v, out_v, enc_v, sem0, sem1, semo0, semo1):
        tabs = (t0h, t1h, t2h, t3h)
        sems = (sem0, sem1)
        semos = (semo0, semo1)
        wid = lax.axis_index("s") * 2 + lax.axis_index("c")
        pltpu.sync_copy(enc_hbm, enc_v)
        pltpu.sync_copy(idx_hbm.at[pl.ds(wid * GPW * S, GPW * S)], idx_v)

        def issue(buf, c):
            off = c * CHUNK_IDX
            for t in range(4):
                for h in range(2):
                    pltpu.async_copy(
                        tabs[t].at[idx_v.at[pl.ds(off + h * HALF, HALF)]],
                        rows_v.at[buf, t, pl.ds(h * HALF, HALF)], sems[buf])

        def drain(buf):
            for t in range(4):
                for h in range(2):
                    pltpu.make_async_copy(
                        tabs[t].at[pl.ds(0, HALF)],
                        rows_v.at[buf, t, pl.ds(h * HALF, HALF)],
                        sems[buf]).wait()

        def drain_out(buf):
            pltpu.make_async_copy(
                out_v.at[buf], out_hbm.at[pl.ds(0, G_CHUNK)],
                semos[buf]).wait()

        def compute_and_write(buf, c):
            for t in range(4):
                e_lo = [enc_v[s, pl.ds(0, 16)] for s in range(S)]
                e_hi = [enc_v[s, pl.ds(16, 16)] for s in range(S)]

                def g_body(g, _):
                    def term(s):
                        w = rows_v[buf, t, g * S + s, :]
                        return plsc.unpack(
                            plsc.bitcast(w, jnp.bfloat16),
                            format=plsc.PackFormat.INTERLEAVED)

                    lo, hi = term(0)
                    acc_lo = lo * e_lo[0]
                    acc_hi = hi * e_hi[0]
                    for s in range(1, S):
                        lo, hi = term(s)
                        acc_lo = acc_lo + lo * e_lo[s]
                        acc_hi = acc_hi + hi * e_hi[s]
                    out_v[buf, g, pl.ds(t * D, 16)] = acc_lo
                    out_v[buf, g, pl.ds(t * D + 16, 16)] = acc_hi
                    return 0

                lax.fori_loop(0, G_CHUNK, g_body, 0, unroll=False)
            gbase = wid * GPW + c * G_CHUNK
            pltpu.async_copy(out_v.at[buf],
                             out_hbm.at[pl.ds(gbase, G_CHUNK)], semos[buf])

        issue(0, 0)

        def loop_body(cc, _):
            c0 = 2 * cc
            issue(1, c0 + 1)
            drain(0)

            @pl.when(cc > 0)
            def _():
                drain_out(0)

            compute_and_write(0, c0)

            @pl.when(cc + 1 < NCH2)
            def _():
                issue(0, c0 + 2)

            drain(1)

            @pl.when(cc > 0)
            def _():
                drain_out(1)

            compute_and_write(1, c0 + 1)
            return 0

        lax.fori_loop(0, NCH2, loop_body, 0, unroll=False)
        drain_out(0)
        drain_out(1)

    return k(t0, t1, t2, t3, idx_all, enc)


def _hops(m_cat, u0):
    BB = 256

    def body(m_ref, u0_ref, u_ref):
        u = u0_ref[...]
        mc = m_ref[...]
        for h in range(HOPS):
            mk = mc[:, :, h * D:(h + 1) * D]
            scores = jnp.sum(mk * u[:, None, :], axis=2)
            smax = jnp.max(scores, axis=1, keepdims=True)
            e = jnp.exp(scores - smax)
            p = e / jnp.sum(e, axis=1, keepdims=True)
            mk1 = mc[:, :, (h + 1) * D:(h + 2) * D]
            u = u + jnp.sum(mk1 * p[:, :, None], axis=1)
        u_ref[...] = u

    return pl.pallas_call(
        body,
        grid=(B // BB,),
        in_specs=[
            pl.BlockSpec((BB, M, DC), lambda i: (i, 0, 0)),
            pl.BlockSpec((BB, D), lambda i: (i, 0)),
        ],
        out_specs=pl.BlockSpec((BB, D), lambda i: (i, 0)),
        out_shape=jax.ShapeDtypeStruct((B, D), jnp.float32),
    )(m_cat, u0)


VSTRIP = 2000
NSTRIP = V // VSTRIP


def _projT_stats(uT, c3):
    def body(c3_ref, uT_ref, ahatT_ref, stat_ref, acc_m, acc_s):
        v = pl.program_id(0)

        @pl.when(v == 0)
        def _():
            acc_m[...] = jnp.full_like(acc_m, -jnp.inf)
            acc_s[...] = jnp.zeros_like(acc_s)

        l = jnp.dot(c3_ref[...], uT_ref[...],
                    preferred_element_type=jnp.float32)
        ahatT_ref[...] = l
        bm = jnp.max(l, axis=0, keepdims=True)
        m_old = acc_m[...]
        m_new = jnp.maximum(m_old, bm)
        acc_s[...] = (acc_s[...] * jnp.exp(m_old - m_new)
                      + jnp.sum(jnp.exp(l - m_new), axis=0, keepdims=True))
        acc_m[...] = m_new

        @pl.when(v == NSTRIP - 1)
        def _():
            stat_ref[0:1, :] = acc_m[...]
            stat_ref[1:2, :] = 1.0 / acc_s[...]

    return pl.pallas_call(
        body,
        grid=(NSTRIP,),
        in_specs=[
            pl.BlockSpec((VSTRIP, D), lambda i: (i, 0)),
            pl.BlockSpec((D, B), lambda i: (0, 0)),
        ],
        out_specs=[
            pl.BlockSpec((VSTRIP, B), lambda i: (i, 0)),
            pl.BlockSpec((8, B), lambda i: (0, 0)),
        ],
        out_shape=[
            jax.ShapeDtypeStruct((V, B), jnp.float32),
            jax.ShapeDtypeStruct((8, B), jnp.float32),
        ],
        scratch_shapes=[
            pltpu.VMEM((1, B), jnp.float32),
            pltpu.VMEM((1, B), jnp.float32),
        ],
    )(c3, uT)


def _projT_write(uT, c3, stats):
    def body(c3_ref, uT_ref, stat_ref, softT_ref):
        l = jnp.dot(c3_ref[...], uT_ref[...],
                    preferred_element_type=jnp.float32)
        m = stat_ref[0:1, :]
        inv_s = stat_ref[1:2, :]
        softT_ref[...] = jnp.exp(l - m) * inv_s

    return pl.pallas_call(
        body,
        grid=(NSTRIP,),
        in_specs=[
            pl.BlockSpec((VSTRIP, D), lambda i: (i, 0)),
            pl.BlockSpec((D, B), lambda i: (0, 0)),
            pl.BlockSpec((8, B), lambda i: (0, 0)),
        ],
        out_specs=pl.BlockSpec((VSTRIP, B), lambda i: (i, 0)),
        out_shape=jax.ShapeDtypeStruct((V, B), jnp.float32),
    )(c3, uT, stats)


def _pack_tables_tc(tables):
    def body(ct_ref, out_ref):
        x = ct_ref[...].astype(jnp.bfloat16)
        lo = jax.lax.bitcast_convert_type(x[:16, :], jnp.uint16)
        hi = jax.lax.bitcast_convert_type(x[16:, :], jnp.uint16)
        w = lo.astype(jnp.uint32) | (hi.astype(jnp.uint32) << 16)
        out_ref[...] = jax.lax.bitcast_convert_type(w, jnp.float32)

    call = pl.pallas_call(
        body,
        grid=(1,),
        in_specs=[pl.BlockSpec((D, V), lambda i: (0, 0))],
        out_specs=pl.BlockSpec((D // 2, V), lambda i: (0, 0)),
        out_shape=jax.ShapeDtypeStruct((D // 2, V), jnp.float32),
    )
    return [call(c.T).T for c in tables]


def kernel(story, query, C0, C1, C2, C3):
    idx_all = jnp.concatenate(
        [story.reshape(-1), query.reshape(-1)]).astype(jnp.int32)
    enc = jnp.asarray(_ENC_CAT[:, :D])

    p0, p1, p2, p3 = _pack_tables_tc([C0, C1, C2, C3])
    m_all = _sc_weighted_gather(p0, p1, p2, p3, idx_all, enc)
    m_cat = m_all[:B * M].reshape(B, M, DC)
    u0 = m_all[B * M:, :D]

    u = _hops(m_cat, u0)
    uT = u.T.astype(jnp.bfloat16)
    c3b = C3.astype(jnp.bfloat16)
    ahatT, stats = _projT_stats(uT, c3b)
    softT = _projT_write(uT, c3b, stats)
    return ahatT.T, softT.T
